# Initial kernel scaffold; baseline (speedup 1.0000x reference)
import jax
import jax.numpy as jnp
from jax.experimental import pallas as pl


def kernel(x, edge_index_in, edge_weight_in, edge_index_out, edge_weight_out, W1_mi, W1_mo, W1_sk, b1_mi, b1_mo, b1_si, b1_so, C1_in, C1_out, W2_mi, W2_mo, W2_sk, b2_mi, b2_mo, b2_si, b2_so, C2_in, C2_out, Wd, bd):
    raise NotImplementedError("write your pallas kernel here")


# speedup vs baseline: 8.4222x; 8.4222x over previous
import math

import jax
import jax.numpy as jnp
import numpy as np
from jax import lax
from jax.experimental import pallas as pl
from jax.experimental.pallas import tpu as pltpu
from jax.experimental.pallas import tpu_sc as plsc

_N = 10000
_E = 320000
_D = 128
_CHUNK = 128
_NCHUNKS = _E // _CHUNK
_NTILES = 16
_ROWS_PER_TILE = 640


def _pe_row():
    div_term = np.exp(np.arange(0, _D, 2, dtype=np.float32) * (-math.log(10000.0) / _D))
    pe = np.zeros((1, _D), dtype=np.float32)
    pe[0, 0::2] = np.sin(0.0 * div_term)
    pe[0, 1::2] = np.cos(0.0 * div_term)
    return jnp.asarray(pe)


def _mm1_body(x_ref, pe_ref, wmi_ref, wmo_ref, wsk_ref, xa_ref, xb_ref):
    h = x_ref[...] + pe_ref[...]
    A = wmi_ref[...] + wsk_ref[...]
    B = wmo_ref[...] + wsk_ref[...]
    dn = (((1,), (1,)), ((), ()))
    xa_ref[...] = lax.dot_general(h, A, dn, preferred_element_type=jnp.float32)
    xb_ref[...] = lax.dot_general(h, B, dn, preferred_element_type=jnp.float32)


def _layer2_body(ic_ref, oc_ref, cin_ref, cout_ref, bmi_ref, bsi_ref,
                 bmo_ref, bso_ref, wmi_ref, wmo_ref, wsk_ref, xa_ref, xb_ref):
    ic = ic_ref[...] + bmi_ref[...] + bsi_ref[...]
    oc = oc_ref[...] + bmo_ref[...] + bso_ref[...]
    h = jnp.tanh(cin_ref[...] * ic + cout_ref[...] * oc)
    A = wmi_ref[...] + wsk_ref[...]
    B = wmo_ref[...] + wsk_ref[...]
    dn = (((1,), (1,)), ((), ()))
    xa_ref[...] = lax.dot_general(h, A, dn, preferred_element_type=jnp.float32)
    xb_ref[...] = lax.dot_general(h, B, dn, preferred_element_type=jnp.float32)


def _decoder_body(ic_ref, oc_ref, cin_ref, cout_ref, bmi_ref, bsi_ref,
                  bmo_ref, bso_ref, wd_ref, bd_ref, logp_ref, emb_ref):
    ic = ic_ref[...] + bmi_ref[...] + bsi_ref[...]
    oc = oc_ref[...] + bmo_ref[...] + bso_ref[...]
    emb = cin_ref[...] * ic + cout_ref[...] * oc
    emb_ref[...] = emb
    ha = jnp.tanh(emb)
    dn = (((1,), (1,)), ((), ()))
    logits = lax.dot_general(ha, wd_ref[...], dn,
                             preferred_element_type=jnp.float32) + bd_ref[...]
    m = jnp.max(logits, axis=1, keepdims=True)
    ls = jnp.log(jnp.sum(jnp.exp(logits - m), axis=1, keepdims=True))
    logp_ref[...] = logits - m - ls


def _prop_body(xa_hbm, xb_hbm, ei_in_hbm, ew_in_hbm, ei_out_hbm, ew_out_hbm,
               zeros_hbm, ic_hbm, oc_hbm,
               acc, idx_src, idx_dst, wbuf, rows, sem):
    c = lax.axis_index("c")
    s = lax.axis_index("s")

    def run(xt_hbm, ei_hbm, ew_hbm, out_hbm):
        @pl.when(s == 0)
        def _():
            pltpu.sync_copy(zeros_hbm, acc)
        plsc.subcore_barrier()

        nch_extra = _NCHUNKS - (_NCHUNKS // _NTILES) * _NTILES
        nch = jnp.where(s < nch_extra,
                        _NCHUNKS // _NTILES + 1, _NCHUNKS // _NTILES)

        def chunk_body(i, carry):
            base = (s + i * _NTILES) * _CHUNK
            pltpu.sync_copy(ei_hbm.at[0, pl.ds(base, _CHUNK)], idx_src)
            pltpu.sync_copy(ei_hbm.at[1, pl.ds(base, _CHUNK)], idx_dst)
            pltpu.sync_copy(ew_hbm.at[pl.ds(base, _CHUNK)], wbuf)
            pltpu.async_copy(xt_hbm.at[idx_src], rows, sem).wait()

            def group_body(g, carry2):
                wv = wbuf[pl.ds(g * 16, 16)]
                for j in range(16):
                    w = wv[j]
                    for d in range(_D // 16):
                        sl = pl.ds(d * 16, 16)
                        rows[g * 16 + j, sl] = rows[g * 16 + j, sl] * w
                return carry2
            lax.fori_loop(0, _CHUNK // 16, group_body, 0, unroll=False)

            pltpu.sync_copy(rows, acc.at[idx_dst], add=True)
            return carry
        lax.fori_loop(0, nch, chunk_body, 0, unroll=False)

        plsc.subcore_barrier()

        @pl.when(s < _NTILES - 1)
        def _():
            rs = pl.ds(s * _ROWS_PER_TILE, _ROWS_PER_TILE)
            pltpu.sync_copy(acc.at[rs], out_hbm.at[rs])

        @pl.when(s == _NTILES - 1)
        def _():
            rs = pl.ds((_NTILES - 1) * _ROWS_PER_TILE,
                       _N - (_NTILES - 1) * _ROWS_PER_TILE)
            pltpu.sync_copy(acc.at[rs], out_hbm.at[rs])

    @pl.when(c == 0)
    def _():
        run(xa_hbm, ei_in_hbm, ew_in_hbm, ic_hbm)

    @pl.when(c == 1)
    def _():
        run(xb_hbm, ei_out_hbm, ew_out_hbm, oc_hbm)


def _make_prop():
    mesh = plsc.VectorSubcoreMesh(core_axis_name="c", subcore_axis_name="s")
    return pl.kernel(
        _prop_body,
        out_type=[jax.ShapeDtypeStruct((_N, _D), jnp.float32),
                  jax.ShapeDtypeStruct((_N, _D), jnp.float32)],
        mesh=mesh,
        scratch_types=[
            pltpu.VMEM_SHARED((_N, _D), jnp.float32),
            pltpu.VMEM((_CHUNK,), jnp.int32),
            pltpu.VMEM((_CHUNK,), jnp.int32),
            pltpu.VMEM((_CHUNK,), jnp.float32),
            pltpu.VMEM((_CHUNK, _D), jnp.float32),
            pltpu.SemaphoreType.DMA,
        ],
    )


_R_MM = 2000
_R_DEC = 200


def kernel(x, edge_index_in, edge_weight_in, edge_index_out, edge_weight_out,
           W1_mi, W1_mo, W1_sk, b1_mi, b1_mo, b1_si, b1_so, C1_in, C1_out,
           W2_mi, W2_mo, W2_sk, b2_mi, b2_mo, b2_si, b2_so, C2_in, C2_out,
           Wd, bd):
    f32 = jnp.float32
    pe0 = _pe_row()
    zeros = jnp.zeros((_N, _D), f32)
    full = lambda shape: pl.BlockSpec(shape, lambda i: (0, 0))
    rowb = lambda r: pl.BlockSpec((r, _D), lambda i: (i, 0))
    colb = lambda r: pl.BlockSpec((r, 1), lambda i: (i, 0))

    b1mi = b1_mi.reshape(1, _D)
    b1si = b1_si.reshape(1, _D)
    b1mo = b1_mo.reshape(1, _D)
    b1so = b1_so.reshape(1, _D)
    b2mi = b2_mi.reshape(1, _D)
    b2si = b2_si.reshape(1, _D)
    b2mo = b2_mo.reshape(1, _D)
    b2so = b2_so.reshape(1, _D)
    bd2 = bd.reshape(1, _N)

    xa1, xb1 = pl.pallas_call(
        _mm1_body,
        grid=(_N // _R_MM,),
        in_specs=[rowb(_R_MM), full((1, _D)), full((_D, _D)), full((_D, _D)),
                  full((_D, _D))],
        out_specs=[rowb(_R_MM), rowb(_R_MM)],
        out_shape=[jax.ShapeDtypeStruct((_N, _D), f32),
                   jax.ShapeDtypeStruct((_N, _D), f32)],
    )(x, pe0, W1_mi, W1_mo, W1_sk)

    prop = _make_prop()
    ic1, oc1 = prop(xa1, xb1, edge_index_in, edge_weight_in,
                    edge_index_out, edge_weight_out, zeros)

    xa2, xb2 = pl.pallas_call(
        _layer2_body,
        grid=(_N // _R_MM,),
        in_specs=[rowb(_R_MM), rowb(_R_MM), colb(_R_MM), colb(_R_MM),
                  full((1, _D)), full((1, _D)), full((1, _D)), full((1, _D)),
                  full((_D, _D)), full((_D, _D)), full((_D, _D))],
        out_specs=[rowb(_R_MM), rowb(_R_MM)],
        out_shape=[jax.ShapeDtypeStruct((_N, _D), f32),
                   jax.ShapeDtypeStruct((_N, _D), f32)],
    )(ic1, oc1, C1_in, C1_out, b1mi, b1si, b1mo, b1so, W2_mi, W2_mo, W2_sk)

    ic2, oc2 = prop(xa2, xb2, edge_index_in, edge_weight_in,
                    edge_index_out, edge_weight_out, zeros)

    logp, emb = pl.pallas_call(
        _decoder_body,
        grid=(_N // _R_DEC,),
        in_specs=[rowb(_R_DEC), rowb(_R_DEC), colb(_R_DEC), colb(_R_DEC),
                  full((1, _D)), full((1, _D)), full((1, _D)), full((1, _D)),
                  full((_N, _D)), full((1, _N))],
        out_specs=[pl.BlockSpec((_R_DEC, _N), lambda i: (i, 0)),
                   rowb(_R_DEC)],
        out_shape=[jax.ShapeDtypeStruct((_N, _N), f32),
                   jax.ShapeDtypeStruct((_N, _D), f32)],
    )(ic2, oc2, C2_in, C2_out, b2mi, b2si, b2mo, b2so, Wd, bd2)

    return (logp, emb)
